# baseline (speedup 1.0000x reference)
import functools

import jax
import jax.numpy as jnp
from jax import lax
from jax.experimental import pallas as pl
from jax.experimental.pallas import tpu as pltpu
from jax.experimental.pallas import tpu_sc as plsc

_NU = 4096
_NI = 16384
_N = _NU + _NI
_D = 128
_DH = 64
_E = 327680
_B = 4096
_NC = 2
_NS = 16
_CH = 128
_EPS = _E // _NS
_NCHUNK = _EPS // _CH
_RPS = _N // _NS
_BB = 2 * _B
_BPS = _BB // _NS
_BCH = _BPS // _CH


def _matmul_body(a_ref, b_ref, bias_ref, o_ref):
    o_ref[...] = (
        jnp.dot(a_ref[...], b_ref[...], preferred_element_type=jnp.float32)
        + bias_ref[...]
    )


def _dense_proj(a, b_t, bias, bm):
    m, k = a.shape
    return pl.pallas_call(
        _matmul_body,
        grid=(m // bm,),
        in_specs=[
            pl.BlockSpec((bm, k), lambda i: (i, 0)),
            pl.BlockSpec((k, _D), lambda i: (0, 0)),
            pl.BlockSpec((1, _D), lambda i: (0, 0)),
        ],
        out_specs=pl.BlockSpec((bm, _D), lambda i: (i, 0)),
        out_shape=jax.ShapeDtypeStruct((m, _D), jnp.float32),
    )(a, b_t, bias)


def _sc_propagate_and_lookup(table, edges_s, idtab_s, idplain_s):
    mesh = plsc.VectorSubcoreMesh(
        core_axis_name="c", subcore_axis_name="s", num_cores=_NC,
        num_subcores=_NS,
    )

    @functools.partial(
        pl.kernel,
        out_type=jax.ShapeDtypeStruct((_NC * _BB, _DH), jnp.float32),
        mesh=mesh,
        compiler_params=pltpu.CompilerParams(
            use_tc_tiling_on_sc=False, needs_layout_passes=False),
        scratch_types=[
            [pltpu.VMEM((3, _CH), jnp.int32)] * 4,
            [pltpu.VMEM((_CH, _DH), jnp.bfloat16)] * 4,
            [pltpu.VMEM((1, _CH), jnp.int32)] * 4,
            pltpu.VMEM((_BCH, _CH), jnp.int32),
            pltpu.VMEM((_BCH, _CH), jnp.int32),
            pltpu.VMEM((_CH, _DH), jnp.float32),
            pltpu.VMEM_SHARED((_N, _DH), jnp.bfloat16),
            [pltpu.SemaphoreType.DMA] * 4,
            [pltpu.SemaphoreType.DMA] * 4,
            [pltpu.SemaphoreType.DMA] * 4,
        ],
    )
    def k(table_hbm, edges_hbm, idtab_hbm, idplain_hbm, out_hbm,
          cbufs, gbufs, sidxs, idtabv, idplainv, fbuf, acc,
          isems, gsems, ssems):
        c = lax.axis_index("c")
        s = lax.axis_index("s")

        pltpu.sync_copy(
            idtab_hbm.at[pl.ds(c * (_NS * _BCH) + s * _BCH, _BCH)], idtabv)
        pltpu.sync_copy(idplain_hbm.at[pl.ds(s * _BCH, _BCH)], idplainv)

        gbuf0, gbuf1 = gbufs[0], gbufs[1]

        @plsc.parallel_loop(0, _CH * (_DH // 32), unroll=4)
        def _zero(j):
            r = j // (_DH // 32)
            o = (j % (_DH // 32)) * 32
            gbuf0[r, pl.ds(o, 32)] = jnp.zeros((32,), jnp.bfloat16)

        for t in range(_RPS // _CH):
            pltpu.sync_copy(gbuf0, acc.at[pl.ds(s * _RPS + t * _CH, _CH)])
        plsc.subcore_barrier()

        ebase = (c * _NS + s) * _NCHUNK

        def start_idx(ci, b):
            pltpu.async_copy(
                edges_hbm.at[pl.ds(3 * (ebase + ci), 3)], cbufs[b], isems[b])

        def wait_idx(ci, b):
            pltpu.make_async_copy(
                edges_hbm.at[pl.ds(3 * (ebase + ci), 3)], cbufs[b],
                isems[b]).wait()

        def start_gather(b):
            pltpu.async_copy(table_hbm.at[cbufs[b].at[0]], gbufs[b], gsems[b])

        def wait_gather(b):
            pltpu.make_async_copy(
                table_hbm.at[cbufs[b].at[0]], gbufs[b], gsems[b]).wait()

        def wait_scatter(b):
            pltpu.make_async_copy(
                gbufs[b], acc.at[sidxs[b].at[0]], ssems[b]).wait()

        for b in range(4):
            start_idx(b, b)
        wait_idx(0, 0)
        start_gather(0)
        wait_idx(1, 1)
        start_gather(1)

        def quad_body(i, carry):
            for b in range(4):
                ci = 4 * i + b
                cb, gb = cbufs[b], gbufs[b]
                wait_gather(b)

                @plsc.parallel_loop(0, _CH // 16, unroll=2)
                def _scale(g):
                    vv = plsc.bitcast(cb[2, pl.ds(g * 16, 16)], jnp.float32)
                    for jj in range(16):
                        vs = jnp.full((16,), vv[jj], jnp.float32)
                        vb = plsc.pack(
                            vs, vs, format=plsc.PackFormat.INTERLEAVED)
                        r = g * 16 + jj
                        for o in range(0, _DH, 32):
                            gb[r, pl.ds(o, 32)] = gb[r, pl.ds(o, 32)] * vb

                for o in range(0, _CH, 16):
                    sidxs[b][0, pl.ds(o, 16)] = cb[1, pl.ds(o, 16)]
                pltpu.async_copy(gb, acc.at[sidxs[b].at[0]], ssems[b],
                                 add=True)

                @pl.when(ci + 4 < _NCHUNK)
                def _():
                    start_idx(ci + 4, b)

                nb = (b + 2) % 4

                @pl.when(ci + 2 < _NCHUNK)
                def _():
                    @pl.when(ci >= 2)
                    def _():
                        wait_scatter(nb)

                    wait_idx(ci + 2, nb)
                    start_gather(nb)

            return carry

        lax.fori_loop(0, _NCHUNK // 4, quad_body, 0)
        for b in range(4):
            wait_scatter(b)
        plsc.subcore_barrier()

        obase = c * _BB + s * _BPS
        for j in range(_BCH):
            pltpu.async_copy(table_hbm.at[idtabv.at[j]], gbuf0, gsems[0])
            pltpu.async_copy(acc.at[idplainv.at[j]], gbuf1, gsems[1])
            pltpu.make_async_copy(
                table_hbm.at[idtabv.at[j]], gbuf0, gsems[0]).wait()
            pltpu.make_async_copy(
                acc.at[idplainv.at[j]], gbuf1, gsems[1]).wait()

            half = jnp.bfloat16(0.5)

            @plsc.parallel_loop(0, _CH, unroll=4)
            def _combine(r):
                for g in range(_DH // 32):
                    sl = pl.ds(g * 32, 32)
                    sm = (gbuf0[r, sl] + gbuf1[r, sl]) * half
                    ev, od = plsc.unpack(
                        sm, format=plsc.PackFormat.INTERLEAVED)
                    fbuf[r, pl.ds(g * 32, 16)] = ev
                    fbuf[r, pl.ds(g * 32 + 16, 16)] = od

            pltpu.sync_copy(fbuf, out_hbm.at[pl.ds(obase + j * _CH, _CH)])

    return k(table, edges_s, idtab_s, idplain_s)


def kernel(u_id, i_id, w_user, w_item, graph_index, graph_values, Wu, bu, Wi, bi):
    u_id = u_id.astype(jnp.int32)
    i_id = i_id.astype(jnp.int32)
    row = graph_index[0].astype(jnp.int32)
    col = graph_index[1].astype(jnp.int32)

    users_emb = _dense_proj(w_user, Wu.T, bu.reshape(1, _D), 512)
    items_emb = _dense_proj(w_item, Wi.T, bi.reshape(1, _D), 1024)
    all_emb = jnp.concatenate([users_emb, items_emb], axis=0)
    table = all_emb.astype(jnp.bfloat16).reshape(_N * 2, _DH)

    vbits = lax.bitcast_convert_type(
        graph_values.astype(jnp.float32), jnp.int32)
    cols_c = 2 * col[None, :] + jnp.arange(_NC, dtype=jnp.int32)[:, None]
    nch = _NS * _NCHUNK
    packed = jnp.stack([
        cols_c.reshape(_NC, nch, _CH),
        jnp.broadcast_to(row.reshape(1, nch, _CH), (_NC, nch, _CH)),
        jnp.broadcast_to(vbits.reshape(1, nch, _CH), (_NC, nch, _CH)),
    ], axis=2)
    edges_s = packed.reshape(_NC * nch * 3, _CH)

    ids = jnp.concatenate([u_id, _NU + i_id])
    idtab_s = (2 * ids[None, :] + jnp.arange(_NC, dtype=jnp.int32)[:, None])
    idtab_s = idtab_s.reshape(_NC * _NS * _BCH, _CH)
    idplain_s = ids.reshape(_NS * _BCH, _CH)

    out = _sc_propagate_and_lookup(table, edges_s, idtab_s, idplain_s)

    halves = out.reshape(_NC, _BB, _DH)
    res = jnp.concatenate([halves[0], halves[1]], axis=1)
    stored = []
    for base in range(0, _D, 32):
        stored.extend(range(base, base + 32, 2))
        stored.extend(range(base + 1, base + 32, 2))
    nat_from_stored = [0] * _D
    for pos, colname in enumerate(stored):
        nat_from_stored[colname] = pos
    res = res[:, jnp.array(nat_from_stored, dtype=jnp.int32)]
    return res[:_B], res[_B:]


# speedup vs baseline: 1.1488x; 1.1488x over previous
import functools

import jax
import jax.numpy as jnp
from jax import lax
from jax.experimental import pallas as pl
from jax.experimental.pallas import tpu as pltpu
from jax.experimental.pallas import tpu_sc as plsc

_NU = 4096
_NI = 16384
_N = _NU + _NI
_D = 128
_DH = 64
_E = 327680
_B = 4096
_NC = 2
_NS = 16
_CH = 128
_KS = 4
_CHE = _KS * _CH
_EPS = _E // _NS
_NCHUNK = _EPS // _CHE
_RPS = _N // _NS
_BB = 2 * _B
_BPS = _BB // _NS
_BCH = _BPS // _CH


def _matmul_body(a_ref, b_ref, bias_ref, o_ref):
    o_ref[...] = (
        jnp.dot(a_ref[...], b_ref[...], preferred_element_type=jnp.float32)
        + bias_ref[...]
    )


def _dense_proj(a, b_t, bias, bm):
    m, k = a.shape
    return pl.pallas_call(
        _matmul_body,
        grid=(m // bm,),
        in_specs=[
            pl.BlockSpec((bm, k), lambda i: (i, 0)),
            pl.BlockSpec((k, _D), lambda i: (0, 0)),
            pl.BlockSpec((1, _D), lambda i: (0, 0)),
        ],
        out_specs=pl.BlockSpec((bm, _D), lambda i: (i, 0)),
        out_shape=jax.ShapeDtypeStruct((m, _D), jnp.float32),
    )(a, b_t, bias)


def _sc_propagate_and_lookup(table, edges_s, idtab_s, idplain_s):
    mesh = plsc.VectorSubcoreMesh(
        core_axis_name="c", subcore_axis_name="s", num_cores=_NC,
        num_subcores=_NS,
    )

    @functools.partial(
        pl.kernel,
        out_type=jax.ShapeDtypeStruct((_NC * _BB, _DH), jnp.float32),
        mesh=mesh,
        compiler_params=pltpu.CompilerParams(
            use_tc_tiling_on_sc=False, needs_layout_passes=False),
        scratch_types=[
            [pltpu.VMEM((3 * _KS, _CH), jnp.int32)] * 4,
            [pltpu.VMEM((_CHE, _DH), jnp.bfloat16)] * 4,
            [pltpu.VMEM((_KS, _CH), jnp.int32)] * 4,
            pltpu.VMEM((_BCH, _CH), jnp.int32),
            pltpu.VMEM((_BCH, _CH), jnp.int32),
            pltpu.VMEM((_CH, _DH), jnp.float32),
            pltpu.VMEM_SHARED((_N, _DH), jnp.bfloat16),
            [pltpu.SemaphoreType.DMA] * 4,
            [pltpu.SemaphoreType.DMA] * 4,
            [pltpu.SemaphoreType.DMA] * 4,
        ],
    )
    def k(table_hbm, edges_hbm, idtab_hbm, idplain_hbm, out_hbm,
          cbufs, gbufs, sidxs, idtabv, idplainv, fbuf, acc,
          isems, gsems, ssems):
        c = lax.axis_index("c")
        s = lax.axis_index("s")

        pltpu.sync_copy(
            idtab_hbm.at[pl.ds(c * (_NS * _BCH) + s * _BCH, _BCH)], idtabv)
        pltpu.sync_copy(idplain_hbm.at[pl.ds(s * _BCH, _BCH)], idplainv)

        gbuf0, gbuf1 = gbufs[0], gbufs[1]

        @plsc.parallel_loop(0, _CHE * (_DH // 32), unroll=4)
        def _zero(j):
            r = j // (_DH // 32)
            o = (j % (_DH // 32)) * 32
            gbuf0[r, pl.ds(o, 32)] = jnp.zeros((32,), jnp.bfloat16)

        for t in range(_RPS // _CHE):
            pltpu.sync_copy(gbuf0, acc.at[pl.ds(s * _RPS + t * _CHE, _CHE)])
        rem = _RPS % _CHE
        if rem:
            pltpu.sync_copy(
                gbuf0.at[pl.ds(0, rem)],
                acc.at[pl.ds(s * _RPS + _RPS - rem, rem)])
        plsc.subcore_barrier()

        ebase = (c * _NS + s) * _NCHUNK

        def start_idx(ci, b):
            pltpu.async_copy(
                edges_hbm.at[pl.ds(3 * _KS * (ebase + ci), 3 * _KS)],
                cbufs[b], isems[b])

        def wait_idx(ci, b):
            pltpu.make_async_copy(
                edges_hbm.at[pl.ds(3 * _KS * (ebase + ci), 3 * _KS)],
                cbufs[b], isems[b]).wait()

        def start_gather(b):
            for q in range(_KS):
                pltpu.async_copy(
                    table_hbm.at[cbufs[b].at[q]],
                    gbufs[b].at[pl.ds(q * _CH, _CH)], gsems[b])

        def wait_gather(b):
            for q in range(_KS):
                pltpu.make_async_copy(
                    table_hbm.at[cbufs[b].at[q]],
                    gbufs[b].at[pl.ds(q * _CH, _CH)], gsems[b]).wait()

        def start_scatter(b):
            for q in range(_KS):
                pltpu.async_copy(
                    gbufs[b].at[pl.ds(q * _CH, _CH)],
                    acc.at[sidxs[b].at[q]], ssems[b], add=True)

        def wait_scatter(b):
            for q in range(_KS):
                pltpu.make_async_copy(
                    gbufs[b].at[pl.ds(q * _CH, _CH)],
                    acc.at[sidxs[b].at[q]], ssems[b]).wait()

        for b in range(4):
            start_idx(b, b)
        wait_idx(0, 0)
        start_gather(0)
        wait_idx(1, 1)
        start_gather(1)

        def quad_body(i, carry):
            for b in range(4):
                ci = 4 * i + b
                cb, gb = cbufs[b], gbufs[b]
                wait_gather(b)

                @plsc.parallel_loop(0, _CHE // 16, unroll=2)
                def _scale(g):
                    vv = plsc.bitcast(
                        cb[2 * _KS + g // 8, pl.ds((g % 8) * 16, 16)],
                        jnp.float32)
                    for jj in range(16):
                        vs = jnp.full((16,), vv[jj], jnp.float32)
                        vb = plsc.pack(
                            vs, vs, format=plsc.PackFormat.INTERLEAVED)
                        r = g * 16 + jj
                        for o in range(0, _DH, 32):
                            gb[r, pl.ds(o, 32)] = gb[r, pl.ds(o, 32)] * vb

                for q in range(_KS):
                    for o in range(0, _CH, 16):
                        sidxs[b][q, pl.ds(o, 16)] = cb[_KS + q, pl.ds(o, 16)]
                start_scatter(b)

                @pl.when(ci + 4 < _NCHUNK)
                def _():
                    start_idx(ci + 4, b)

                nb = (b + 2) % 4

                @pl.when(ci + 2 < _NCHUNK)
                def _():
                    @pl.when(ci >= 2)
                    def _():
                        wait_scatter(nb)

                    wait_idx(ci + 2, nb)
                    start_gather(nb)

            return carry

        lax.fori_loop(0, _NCHUNK // 4, quad_body, 0)
        for b in range(4):
            wait_scatter(b)
        plsc.subcore_barrier()

        obase = c * _BB + s * _BPS
        gb0v = gbuf0.at[pl.ds(0, _CH)]
        gb1v = gbuf1.at[pl.ds(0, _CH)]
        for j in range(_BCH):
            pltpu.async_copy(table_hbm.at[idtabv.at[j]], gb0v, gsems[0])
            pltpu.async_copy(acc.at[idplainv.at[j]], gb1v, gsems[1])
            pltpu.make_async_copy(
                table_hbm.at[idtabv.at[j]], gb0v, gsems[0]).wait()
            pltpu.make_async_copy(
                acc.at[idplainv.at[j]], gb1v, gsems[1]).wait()

            half = jnp.bfloat16(0.5)

            @plsc.parallel_loop(0, _CH, unroll=4)
            def _combine(r):
                for g in range(_DH // 32):
                    sl = pl.ds(g * 32, 32)
                    sm = (gbuf0[r, sl] + gbuf1[r, sl]) * half
                    ev, od = plsc.unpack(
                        sm, format=plsc.PackFormat.INTERLEAVED)
                    fbuf[r, pl.ds(g * 32, 16)] = ev
                    fbuf[r, pl.ds(g * 32 + 16, 16)] = od

            pltpu.sync_copy(fbuf, out_hbm.at[pl.ds(obase + j * _CH, _CH)])

    return k(table, edges_s, idtab_s, idplain_s)


def kernel(u_id, i_id, w_user, w_item, graph_index, graph_values, Wu, bu, Wi, bi):
    u_id = u_id.astype(jnp.int32)
    i_id = i_id.astype(jnp.int32)
    row = graph_index[0].astype(jnp.int32)
    col = graph_index[1].astype(jnp.int32)

    users_emb = _dense_proj(w_user, Wu.T, bu.reshape(1, _D), 512)
    items_emb = _dense_proj(w_item, Wi.T, bi.reshape(1, _D), 1024)
    all_emb = jnp.concatenate([users_emb, items_emb], axis=0)
    table = all_emb.astype(jnp.bfloat16).reshape(_N * 2, _DH)

    vbits = lax.bitcast_convert_type(
        graph_values.astype(jnp.float32), jnp.int32)
    cols_c = 2 * col[None, :] + jnp.arange(_NC, dtype=jnp.int32)[:, None]
    nch = _NS * _NCHUNK
    packed = jnp.stack([
        cols_c.reshape(_NC, nch, _KS, _CH),
        jnp.broadcast_to(
            row.reshape(1, nch, _KS, _CH), (_NC, nch, _KS, _CH)),
        jnp.broadcast_to(
            vbits.reshape(1, nch, _KS, _CH), (_NC, nch, _KS, _CH)),
    ], axis=2)
    edges_s = packed.reshape(_NC * nch * 3 * _KS, _CH)

    ids = jnp.concatenate([u_id, _NU + i_id])
    idtab_s = (2 * ids[None, :] + jnp.arange(_NC, dtype=jnp.int32)[:, None])
    idtab_s = idtab_s.reshape(_NC * _NS * _BCH, _CH)
    idplain_s = ids.reshape(_NS * _BCH, _CH)

    out = _sc_propagate_and_lookup(table, edges_s, idtab_s, idplain_s)

    halves = out.reshape(_NC, _BB, _DH)
    res = jnp.concatenate([halves[0], halves[1]], axis=1)
    stored = []
    for base in range(0, _D, 32):
        stored.extend(range(base, base + 32, 2))
        stored.extend(range(base + 1, base + 32, 2))
    nat_from_stored = [0] * _D
    for pos, colname in enumerate(stored):
        nat_from_stored[colname] = pos
    res = res[:, jnp.array(nat_from_stored, dtype=jnp.int32)]
    return res[:_B], res[_B:]


# speedup vs baseline: 1.1653x; 1.0144x over previous
import functools

import jax
import jax.numpy as jnp
from jax import lax
from jax.experimental import pallas as pl
from jax.experimental.pallas import tpu as pltpu
from jax.experimental.pallas import tpu_sc as plsc

_NU = 4096
_NI = 16384
_N = _NU + _NI
_D = 128
_DH = 64
_E = 327680
_B = 4096
_NC = 2
_NS = 16
_CH = 128
_KS = 4
_CHE = _KS * _CH
_EPS = _E // _NS
_NCHUNK = _EPS // _CHE
_RPS = _N // _NS
_BB = 2 * _B
_BPS = _BB // _NS
_BCH = _BPS // _CH


def _matmul_body(a_ref, b_ref, bias_ref, o_ref):
    o_ref[...] = (
        jnp.dot(a_ref[...], b_ref[...], preferred_element_type=jnp.float32)
        + bias_ref[...]
    )


def _dense_proj(a, b_t, bias, bm):
    m, k = a.shape
    return pl.pallas_call(
        _matmul_body,
        grid=(m // bm,),
        in_specs=[
            pl.BlockSpec((bm, k), lambda i: (i, 0)),
            pl.BlockSpec((k, _D), lambda i: (0, 0)),
            pl.BlockSpec((1, _D), lambda i: (0, 0)),
        ],
        out_specs=pl.BlockSpec((bm, _D), lambda i: (i, 0)),
        out_shape=jax.ShapeDtypeStruct((m, _D), jnp.float32),
    )(a, b_t, bias)


def _sc_propagate_and_lookup(table, edges_s, idtab_s, idplain_s):
    mesh = plsc.VectorSubcoreMesh(
        core_axis_name="c", subcore_axis_name="s", num_cores=_NC,
        num_subcores=_NS,
    )

    @functools.partial(
        pl.kernel,
        out_type=jax.ShapeDtypeStruct((_NC * _BB, _DH), jnp.float32),
        mesh=mesh,
        compiler_params=pltpu.CompilerParams(
            use_tc_tiling_on_sc=False, needs_layout_passes=False),
        scratch_types=[
            [pltpu.VMEM((3 * _KS, _CH), jnp.int32)] * 4,
            [pltpu.VMEM((_CHE, _DH), jnp.bfloat16)] * 4,
            [pltpu.VMEM((_KS, _CH), jnp.int32)] * 4,
            pltpu.VMEM((_BCH, _CH), jnp.int32),
            pltpu.VMEM((_BCH, _CH), jnp.int32),
            pltpu.VMEM((_CH, _DH), jnp.float32),
            pltpu.VMEM_SHARED((_N, _DH), jnp.bfloat16),
            [pltpu.SemaphoreType.DMA] * 4,
            [pltpu.SemaphoreType.DMA] * 4,
            [pltpu.SemaphoreType.DMA] * 4,
        ],
    )
    def k(table_hbm, edges_hbm, idtab_hbm, idplain_hbm, out_hbm,
          cbufs, gbufs, sidxs, idtabv, idplainv, fbuf, acc,
          isems, gsems, ssems):
        c = lax.axis_index("c")
        s = lax.axis_index("s")

        pltpu.sync_copy(
            idtab_hbm.at[pl.ds(c * (_NS * _BCH) + s * _BCH, _BCH)], idtabv)
        pltpu.sync_copy(idplain_hbm.at[pl.ds(s * _BCH, _BCH)], idplainv)

        gbuf0, gbuf1 = gbufs[0], gbufs[1]
        zbuf = gbufs[3]

        ebase = (c * _NS + s) * _NCHUNK

        def start_idx(ci, b):
            pltpu.async_copy(
                edges_hbm.at[pl.ds(3 * _KS * (ebase + ci), 3 * _KS)],
                cbufs[b], isems[b])

        def wait_idx(ci, b):
            pltpu.make_async_copy(
                edges_hbm.at[pl.ds(3 * _KS * (ebase + ci), 3 * _KS)],
                cbufs[b], isems[b]).wait()

        def start_gather(b):
            for q in range(_KS):
                pltpu.async_copy(
                    table_hbm.at[cbufs[b].at[q]],
                    gbufs[b].at[pl.ds(q * _CH, _CH)], gsems[b])

        def wait_gather(b):
            for q in range(_KS):
                pltpu.make_async_copy(
                    table_hbm.at[cbufs[b].at[q]],
                    gbufs[b].at[pl.ds(q * _CH, _CH)], gsems[b]).wait()

        def start_scatter(b):
            for q in range(_KS):
                pltpu.async_copy(
                    gbufs[b].at[pl.ds(q * _CH, _CH)],
                    acc.at[sidxs[b].at[q]], ssems[b], add=True)

        def wait_scatter(b):
            for q in range(_KS):
                pltpu.make_async_copy(
                    gbufs[b].at[pl.ds(q * _CH, _CH)],
                    acc.at[sidxs[b].at[q]], ssems[b]).wait()

        for b in range(4):
            start_idx(b, b)
        wait_idx(0, 0)
        start_gather(0)
        wait_idx(1, 1)
        start_gather(1)

        @plsc.parallel_loop(0, _CHE * (_DH // 32), unroll=4)
        def _zero(j):
            r = j // (_DH // 32)
            o = (j % (_DH // 32)) * 32
            zbuf[r, pl.ds(o, 32)] = jnp.zeros((32,), jnp.bfloat16)

        for t in range(_RPS // _CHE):
            pltpu.sync_copy(zbuf, acc.at[pl.ds(s * _RPS + t * _CHE, _CHE)])
        rem = _RPS % _CHE
        if rem:
            pltpu.sync_copy(
                zbuf.at[pl.ds(0, rem)],
                acc.at[pl.ds(s * _RPS + _RPS - rem, rem)])
        plsc.subcore_barrier()

        def quad_body(i, carry):
            for b in range(4):
                ci = 4 * i + b
                cb, gb = cbufs[b], gbufs[b]
                wait_gather(b)

                @plsc.parallel_loop(0, _CHE // 16, unroll=2)
                def _scale(g):
                    vv = plsc.bitcast(
                        cb[2 * _KS + g // 8, pl.ds((g % 8) * 16, 16)],
                        jnp.float32)
                    for jj in range(16):
                        vs = jnp.full((16,), vv[jj], jnp.float32)
                        vb = plsc.pack(
                            vs, vs, format=plsc.PackFormat.INTERLEAVED)
                        r = g * 16 + jj
                        for o in range(0, _DH, 32):
                            gb[r, pl.ds(o, 32)] = gb[r, pl.ds(o, 32)] * vb

                for q in range(_KS):
                    for o in range(0, _CH, 16):
                        sidxs[b][q, pl.ds(o, 16)] = cb[_KS + q, pl.ds(o, 16)]
                start_scatter(b)

                @pl.when(ci + 4 < _NCHUNK)
                def _():
                    start_idx(ci + 4, b)

                nb = (b + 2) % 4

                @pl.when(ci + 2 < _NCHUNK)
                def _():
                    @pl.when(ci >= 2)
                    def _():
                        wait_scatter(nb)

                    wait_idx(ci + 2, nb)
                    start_gather(nb)

            return carry

        lax.fori_loop(0, _NCHUNK // 4, quad_body, 0)
        for b in range(4):
            wait_scatter(b)
        plsc.subcore_barrier()

        obase = c * _BB + s * _BPS
        pairs = (
            (gbufs[0], gbufs[1], gsems[0], gsems[1]),
            (gbufs[2], gbufs[3], gsems[2], gsems[3]),
        )

        def lk_start(j, p):
            tb, ab, tsem, asem = pairs[p]
            pltpu.async_copy(
                table_hbm.at[idtabv.at[j]], tb.at[pl.ds(0, _CH)], tsem)
            pltpu.async_copy(
                acc.at[idplainv.at[j]], ab.at[pl.ds(0, _CH)], asem)

        def lk_wait(j, p):
            tb, ab, tsem, asem = pairs[p]
            pltpu.make_async_copy(
                table_hbm.at[idtabv.at[j]], tb.at[pl.ds(0, _CH)], tsem).wait()
            pltpu.make_async_copy(
                acc.at[idplainv.at[j]], ab.at[pl.ds(0, _CH)], asem).wait()

        half = jnp.bfloat16(0.5)
        lk_start(0, 0)
        for j in range(_BCH):
            if j + 1 < _BCH:
                lk_start(j + 1, (j + 1) % 2)
            lk_wait(j, j % 2)
            tb, ab, _, _ = pairs[j % 2]

            @plsc.parallel_loop(0, _CH, unroll=4)
            def _combine(r):
                for g in range(_DH // 32):
                    sl = pl.ds(g * 32, 32)
                    sm = (tb[r, sl] + ab[r, sl]) * half
                    ev, od = plsc.unpack(
                        sm, format=plsc.PackFormat.INTERLEAVED)
                    fbuf[r, pl.ds(g * 32, 16)] = ev
                    fbuf[r, pl.ds(g * 32 + 16, 16)] = od

            pltpu.sync_copy(fbuf, out_hbm.at[pl.ds(obase + j * _CH, _CH)])

    return k(table, edges_s, idtab_s, idplain_s)


def kernel(u_id, i_id, w_user, w_item, graph_index, graph_values, Wu, bu, Wi, bi):
    u_id = u_id.astype(jnp.int32)
    i_id = i_id.astype(jnp.int32)
    row = graph_index[0].astype(jnp.int32)
    col = graph_index[1].astype(jnp.int32)

    users_emb = _dense_proj(w_user, Wu.T, bu.reshape(1, _D), 512)
    items_emb = _dense_proj(w_item, Wi.T, bi.reshape(1, _D), 1024)
    all_emb = jnp.concatenate([users_emb, items_emb], axis=0)
    table = all_emb.astype(jnp.bfloat16).reshape(_N * 2, _DH)

    vbits = lax.bitcast_convert_type(
        graph_values.astype(jnp.float32), jnp.int32)
    cols_c = 2 * col[None, :] + jnp.arange(_NC, dtype=jnp.int32)[:, None]
    nch = _NS * _NCHUNK
    packed = jnp.stack([
        cols_c.reshape(_NC, nch, _KS, _CH),
        jnp.broadcast_to(
            row.reshape(1, nch, _KS, _CH), (_NC, nch, _KS, _CH)),
        jnp.broadcast_to(
            vbits.reshape(1, nch, _KS, _CH), (_NC, nch, _KS, _CH)),
    ], axis=2)
    edges_s = packed.reshape(_NC * nch * 3 * _KS, _CH)

    ids = jnp.concatenate([u_id, _NU + i_id])
    idtab_s = (2 * ids[None, :] + jnp.arange(_NC, dtype=jnp.int32)[:, None])
    idtab_s = idtab_s.reshape(_NC * _NS * _BCH, _CH)
    idplain_s = ids.reshape(_NS * _BCH, _CH)

    out = _sc_propagate_and_lookup(table, edges_s, idtab_s, idplain_s)

    halves = out.reshape(_NC, _BB, _DH)
    res = jnp.concatenate([halves[0], halves[1]], axis=1)
    stored = []
    for base in range(0, _D, 32):
        stored.extend(range(base, base + 32, 2))
        stored.extend(range(base + 1, base + 32, 2))
    nat_from_stored = [0] * _D
    for pos, colname in enumerate(stored):
        nat_from_stored[colname] = pos
    res = res[:, jnp.array(nat_from_stored, dtype=jnp.int32)]
    return res[:_B], res[_B:]


# speedup vs baseline: 1.1714x; 1.0052x over previous
import functools

import jax
import jax.numpy as jnp
from jax import lax
from jax.experimental import pallas as pl
from jax.experimental.pallas import tpu as pltpu
from jax.experimental.pallas import tpu_sc as plsc

_NU = 4096
_NI = 16384
_N = _NU + _NI
_D = 128
_DH = 64
_E = 327680
_B = 4096
_NC = 2
_NS = 16
_CH = 128
_KS = 4
_CHE = _KS * _CH
_EPS = _E // _NS
_NCHUNK = _EPS // _CHE
_RPS = _N // _NS
_BB = 2 * _B
_BPS = _BB // _NS
_BCH = _BPS // _CH


def _matmul_body(a_ref, b_ref, bias_ref, o_ref):
    o_ref[...] = (
        jnp.dot(a_ref[...], b_ref[...], preferred_element_type=jnp.float32)
        + bias_ref[...]
    )


def _dense_proj(a, b_t, bias, bm):
    m, k = a.shape
    return pl.pallas_call(
        _matmul_body,
        grid=(m // bm,),
        in_specs=[
            pl.BlockSpec((bm, k), lambda i: (i, 0)),
            pl.BlockSpec((k, _D), lambda i: (0, 0)),
            pl.BlockSpec((1, _D), lambda i: (0, 0)),
        ],
        out_specs=pl.BlockSpec((bm, _D), lambda i: (i, 0)),
        out_shape=jax.ShapeDtypeStruct((m, _D), jnp.float32),
    )(a, b_t, bias)


def _sc_propagate_and_lookup(table, edges_s, idtab_s, idplain_s):
    mesh = plsc.VectorSubcoreMesh(
        core_axis_name="c", subcore_axis_name="s", num_cores=_NC,
        num_subcores=_NS,
    )

    @functools.partial(
        pl.kernel,
        out_type=jax.ShapeDtypeStruct((_NC * _BB, _DH), jnp.float32),
        mesh=mesh,
        compiler_params=pltpu.CompilerParams(
            use_tc_tiling_on_sc=False, needs_layout_passes=False),
        scratch_types=[
            [pltpu.VMEM((3 * _KS, _CH), jnp.int32)] * 4,
            [pltpu.VMEM((_CHE, _DH), jnp.bfloat16)] * 4,
            [pltpu.VMEM((_KS, _CH), jnp.int32)] * 4,
            pltpu.VMEM((_BCH, _CH), jnp.int32),
            pltpu.VMEM((_BCH, _CH), jnp.int32),
            pltpu.VMEM((_CH, _DH), jnp.float32),
            pltpu.VMEM_SHARED((_N, _DH), jnp.bfloat16),
            [pltpu.SemaphoreType.DMA] * 4,
            [pltpu.SemaphoreType.DMA] * 4,
            [pltpu.SemaphoreType.DMA] * 4,
        ],
    )
    def k(table_hbm, edges_hbm, idtab_hbm, idplain_hbm, out_hbm,
          cbufs, gbufs, sidxs, idtabv, idplainv, fbuf, acc,
          isems, gsems, ssems):
        c = lax.axis_index("c")
        s = lax.axis_index("s")

        pltpu.sync_copy(
            idtab_hbm.at[pl.ds(c * (_NS * _BCH) + s * _BCH, _BCH)], idtabv)
        pltpu.sync_copy(idplain_hbm.at[pl.ds(s * _BCH, _BCH)], idplainv)

        gbuf0, gbuf1 = gbufs[0], gbufs[1]
        zbuf = gbufs[3]

        ebase = (c * _NS + s) * _NCHUNK

        def start_idx(ci, b):
            pltpu.async_copy(
                edges_hbm.at[pl.ds(3 * _KS * (ebase + ci), 3 * _KS)],
                cbufs[b], isems[b])

        def wait_idx(ci, b):
            pltpu.make_async_copy(
                edges_hbm.at[pl.ds(3 * _KS * (ebase + ci), 3 * _KS)],
                cbufs[b], isems[b]).wait()

        def start_gather(b):
            for q in range(_KS):
                pltpu.async_copy(
                    table_hbm.at[cbufs[b].at[q]],
                    gbufs[b].at[pl.ds(q * _CH, _CH)], gsems[b])

        def wait_gather(b):
            for q in range(_KS):
                pltpu.make_async_copy(
                    table_hbm.at[cbufs[b].at[q]],
                    gbufs[b].at[pl.ds(q * _CH, _CH)], gsems[b]).wait()

        def start_scatter(b):
            for q in range(_KS):
                pltpu.async_copy(
                    gbufs[b].at[pl.ds(q * _CH, _CH)],
                    acc.at[sidxs[b].at[q]], ssems[b], add=True)

        def wait_scatter(b):
            for q in range(_KS):
                pltpu.make_async_copy(
                    gbufs[b].at[pl.ds(q * _CH, _CH)],
                    acc.at[sidxs[b].at[q]], ssems[b]).wait()

        for b in range(4):
            start_idx(b, b)
        wait_idx(0, 0)
        start_gather(0)
        wait_idx(1, 1)
        start_gather(1)

        @plsc.parallel_loop(0, _CHE * (_DH // 32), unroll=4)
        def _zero(j):
            r = j // (_DH // 32)
            o = (j % (_DH // 32)) * 32
            zbuf[r, pl.ds(o, 32)] = jnp.zeros((32,), jnp.bfloat16)

        for t in range(_RPS // _CHE):
            pltpu.sync_copy(zbuf, acc.at[pl.ds(s * _RPS + t * _CHE, _CHE)])
        rem = _RPS % _CHE
        if rem:
            pltpu.sync_copy(
                zbuf.at[pl.ds(0, rem)],
                acc.at[pl.ds(s * _RPS + _RPS - rem, rem)])
        plsc.subcore_barrier()

        def quad_body(i, carry):
            for b in range(4):
                ci = 4 * i + b
                cb, gb = cbufs[b], gbufs[b]
                wait_gather(b)

                @plsc.parallel_loop(0, _CHE // 16, unroll=2)
                def _scale(g):
                    vv = plsc.bitcast(
                        cb[2 * _KS + g // 8, pl.ds((g % 8) * 16, 16)],
                        jnp.float32)
                    for jj in range(16):
                        vs = jnp.full((16,), vv[jj], jnp.float32)
                        vb = plsc.pack(
                            vs, vs, format=plsc.PackFormat.INTERLEAVED)
                        r = g * 16 + jj
                        for o in range(0, _DH, 32):
                            gb[r, pl.ds(o, 32)] = gb[r, pl.ds(o, 32)] * vb

                for q in range(_KS):
                    for o in range(0, _CH, 16):
                        sidxs[b][q, pl.ds(o, 16)] = cb[_KS + q, pl.ds(o, 16)]
                start_scatter(b)

                @pl.when(ci + 4 < _NCHUNK)
                def _():
                    start_idx(ci + 4, b)

                nb = (b + 2) % 4

                @pl.when(ci + 2 < _NCHUNK)
                def _():
                    @pl.when(ci >= 2)
                    def _():
                        wait_scatter(nb)

                    wait_idx(ci + 2, nb)
                    start_gather(nb)

            return carry

        lax.fori_loop(0, _NCHUNK // 4, quad_body, 0)
        for b in range(4):
            wait_scatter(b)
        plsc.subcore_barrier()

        obase = c * _BB + s * _BPS
        pairs = (
            (gbufs[0], gbufs[1], gsems[0], gsems[1]),
            (gbufs[2], gbufs[3], gsems[2], gsems[3]),
        )

        def lk_start(j, p):
            tb, ab, tsem, asem = pairs[p]
            pltpu.async_copy(
                table_hbm.at[idtabv.at[j]], tb.at[pl.ds(0, _CH)], tsem)
            pltpu.async_copy(
                acc.at[idplainv.at[j]], ab.at[pl.ds(0, _CH)], asem)

        def lk_wait(j, p):
            tb, ab, tsem, asem = pairs[p]
            pltpu.make_async_copy(
                table_hbm.at[idtabv.at[j]], tb.at[pl.ds(0, _CH)], tsem).wait()
            pltpu.make_async_copy(
                acc.at[idplainv.at[j]], ab.at[pl.ds(0, _CH)], asem).wait()

        half = jnp.bfloat16(0.5)
        lk_start(0, 0)
        for j in range(_BCH):
            if j + 1 < _BCH:
                lk_start(j + 1, (j + 1) % 2)
            lk_wait(j, j % 2)
            tb, ab, _, _ = pairs[j % 2]

            @plsc.parallel_loop(0, _CH, unroll=4)
            def _combine(r):
                for g in range(_DH // 32):
                    sl = pl.ds(g * 32, 32)
                    sm = (tb[r, sl] + ab[r, sl]) * half
                    ev, od = plsc.unpack(
                        sm, format=plsc.PackFormat.INTERLEAVED)
                    fbuf[r, pl.ds(g * 32, 16)] = ev
                    fbuf[r, pl.ds(g * 32 + 16, 16)] = od

            pltpu.sync_copy(fbuf, out_hbm.at[pl.ds(obase + j * _CH, _CH)])

    return k(table, edges_s, idtab_s, idplain_s)


def kernel(u_id, i_id, w_user, w_item, graph_index, graph_values, Wu, bu, Wi, bi):
    u_id = u_id.astype(jnp.int32)
    i_id = i_id.astype(jnp.int32)
    row = graph_index[0].astype(jnp.int32)
    col = graph_index[1].astype(jnp.int32)

    users_emb = _dense_proj(w_user, Wu.T, bu.reshape(1, _D), 1024)
    items_emb = _dense_proj(w_item, Wi.T, bi.reshape(1, _D), 2048)
    all_emb = jnp.concatenate([users_emb, items_emb], axis=0)
    table = all_emb.astype(jnp.bfloat16).reshape(_N * 2, _DH)

    vbits = lax.bitcast_convert_type(
        graph_values.astype(jnp.float32), jnp.int32)
    cols_c = 2 * col[None, :] + jnp.arange(_NC, dtype=jnp.int32)[:, None]
    nch = _NS * _NCHUNK
    packed = jnp.stack([
        cols_c.reshape(_NC, nch, _KS, _CH),
        jnp.broadcast_to(
            row.reshape(1, nch, _KS, _CH), (_NC, nch, _KS, _CH)),
        jnp.broadcast_to(
            vbits.reshape(1, nch, _KS, _CH), (_NC, nch, _KS, _CH)),
    ], axis=2)
    edges_s = packed.reshape(_NC * nch * 3 * _KS, _CH)

    ids = jnp.concatenate([u_id, _NU + i_id])
    idtab_s = (2 * ids[None, :] + jnp.arange(_NC, dtype=jnp.int32)[:, None])
    idtab_s = idtab_s.reshape(_NC * _NS * _BCH, _CH)
    idplain_s = ids.reshape(_NS * _BCH, _CH)

    out = _sc_propagate_and_lookup(table, edges_s, idtab_s, idplain_s)

    halves = out.reshape(_NC, _BB, _DH)
    res = jnp.concatenate([halves[0], halves[1]], axis=1)
    stored = []
    for base in range(0, _D, 32):
        stored.extend(range(base, base + 32, 2))
        stored.extend(range(base + 1, base + 32, 2))
    nat_from_stored = [0] * _D
    for pos, colname in enumerate(stored):
        nat_from_stored[colname] = pos
    res = res[:, jnp.array(nat_from_stored, dtype=jnp.int32)]
    return res[:_B], res[_B:]


# speedup vs baseline: 1.1849x; 1.0115x over previous
import functools

import jax
import jax.numpy as jnp
from jax import lax
from jax.experimental import pallas as pl
from jax.experimental.pallas import tpu as pltpu
from jax.experimental.pallas import tpu_sc as plsc

_NU = 4096
_NI = 16384
_N = _NU + _NI
_D = 128
_DH = 64
_E = 327680
_B = 4096
_NC = 2
_NS = 16
_CH = 128
_KS = 4
_CHE = _KS * _CH
_EPS = _E // _NS
_NCHUNK = _EPS // _CHE
_RPS = _N // _NS
_BB = 2 * _B
_BPS = _BB // _NS
_BCH = _BPS // _CH


def _matmul_body(a_ref, b_ref, bias_ref, o_ref):
    o_ref[...] = (
        jnp.dot(a_ref[...], b_ref[...], preferred_element_type=jnp.float32)
        + bias_ref[...]
    )


def _dense_proj(a, b_t, bias, bm):
    m, k = a.shape
    return pl.pallas_call(
        _matmul_body,
        grid=(m // bm,),
        in_specs=[
            pl.BlockSpec((bm, k), lambda i: (i, 0)),
            pl.BlockSpec((k, _D), lambda i: (0, 0)),
            pl.BlockSpec((1, _D), lambda i: (0, 0)),
        ],
        out_specs=pl.BlockSpec((bm, _D), lambda i: (i, 0)),
        out_shape=jax.ShapeDtypeStruct((m, _D), jnp.float32),
    )(a, b_t, bias)


def _sc_propagate_and_lookup(table, edges_s, idtab_s, idplain_s):
    mesh = plsc.VectorSubcoreMesh(
        core_axis_name="c", subcore_axis_name="s", num_cores=_NC,
        num_subcores=_NS,
    )

    @functools.partial(
        pl.kernel,
        out_type=jax.ShapeDtypeStruct((_NC * _BB, _DH), jnp.float32),
        mesh=mesh,
        compiler_params=pltpu.CompilerParams(
            use_tc_tiling_on_sc=False, needs_layout_passes=False),
        scratch_types=[
            [pltpu.VMEM((3 * _KS, _CH), jnp.int32)] * 4,
            [pltpu.VMEM((_CHE, _DH), jnp.bfloat16)] * 4,
            [pltpu.VMEM((_KS, _CH), jnp.int32)] * 4,
            pltpu.VMEM((_BCH, _CH), jnp.int32),
            pltpu.VMEM((_BCH, _CH), jnp.int32),
            pltpu.VMEM((_CH, _DH), jnp.float32),
            pltpu.VMEM_SHARED((_N, _DH), jnp.bfloat16),
            [pltpu.SemaphoreType.DMA] * 4,
            [pltpu.SemaphoreType.DMA] * 4,
            [pltpu.SemaphoreType.DMA] * 4,
            [pltpu.SemaphoreType.DMA] * 2,
        ],
    )
    def k(table_hbm, edges_hbm, idtab_hbm, idplain_hbm, out_hbm,
          cbufs, gbufs, sidxs, idtabv, idplainv, fbuf, acc,
          isems, gsems, ssems, xsems):
        c = lax.axis_index("c")
        s = lax.axis_index("s")

        idtab_src = idtab_hbm.at[pl.ds(c * (_NS * _BCH) + s * _BCH, _BCH)]
        idplain_src = idplain_hbm.at[pl.ds(s * _BCH, _BCH)]
        pltpu.async_copy(idtab_src, idtabv, xsems[0])
        pltpu.async_copy(idplain_src, idplainv, xsems[1])

        gbuf0, gbuf1 = gbufs[0], gbufs[1]
        zbuf = gbufs[3]

        ebase = s * _NCHUNK

        def start_idx(ci, b):
            pltpu.async_copy(
                edges_hbm.at[pl.ds(3 * _KS * (ebase + ci), 3 * _KS)],
                cbufs[b], isems[b])

        def wait_idx(ci, b):
            pltpu.make_async_copy(
                edges_hbm.at[pl.ds(3 * _KS * (ebase + ci), 3 * _KS)],
                cbufs[b], isems[b]).wait()

        def fix_cols(b):
            cb = cbufs[b]
            for q in range(_KS):
                for o in range(0, _CH, 16):
                    sl = pl.ds(o, 16)
                    cb[q, sl] = cb[q, sl] * 2 + c

        def start_gather(b):
            for q in range(_KS):
                pltpu.async_copy(
                    table_hbm.at[cbufs[b].at[q]],
                    gbufs[b].at[pl.ds(q * _CH, _CH)], gsems[b])

        def wait_gather(b):
            for q in range(_KS):
                pltpu.make_async_copy(
                    table_hbm.at[cbufs[b].at[q]],
                    gbufs[b].at[pl.ds(q * _CH, _CH)], gsems[b]).wait()

        def start_scatter(b):
            for q in range(_KS):
                pltpu.async_copy(
                    gbufs[b].at[pl.ds(q * _CH, _CH)],
                    acc.at[sidxs[b].at[q]], ssems[b], add=True)

        def wait_scatter(b):
            for q in range(_KS):
                pltpu.make_async_copy(
                    gbufs[b].at[pl.ds(q * _CH, _CH)],
                    acc.at[sidxs[b].at[q]], ssems[b]).wait()

        for b in range(4):
            start_idx(b, b)
        wait_idx(0, 0)
        fix_cols(0)
        start_gather(0)
        wait_idx(1, 1)
        fix_cols(1)
        start_gather(1)

        @plsc.parallel_loop(0, _CHE * (_DH // 32), unroll=4)
        def _zero(j):
            r = j // (_DH // 32)
            o = (j % (_DH // 32)) * 32
            zbuf[r, pl.ds(o, 32)] = jnp.zeros((32,), jnp.bfloat16)

        nz = _RPS // _CHE
        for t in range(nz):
            pltpu.async_copy(
                zbuf, acc.at[pl.ds(s * _RPS + t * _CHE, _CHE)], ssems[t])
        rem = _RPS % _CHE
        if rem:
            pltpu.async_copy(
                zbuf.at[pl.ds(0, rem)],
                acc.at[pl.ds(s * _RPS + _RPS - rem, rem)], ssems[nz])
        for t in range(nz):
            pltpu.make_async_copy(
                zbuf, acc.at[pl.ds(s * _RPS + t * _CHE, _CHE)],
                ssems[t]).wait()
        if rem:
            pltpu.make_async_copy(
                zbuf.at[pl.ds(0, rem)],
                acc.at[pl.ds(s * _RPS + _RPS - rem, rem)], ssems[nz]).wait()
        plsc.subcore_barrier()

        def quad_body(i, carry):
            for b in range(4):
                ci = 4 * i + b
                cb, gb = cbufs[b], gbufs[b]
                wait_gather(b)

                @plsc.parallel_loop(0, _CHE // 16, unroll=2)
                def _scale(g):
                    vv = plsc.bitcast(
                        cb[2 * _KS + g // 8, pl.ds((g % 8) * 16, 16)],
                        jnp.float32)
                    for jj in range(16):
                        vs = jnp.full((16,), vv[jj], jnp.float32)
                        vb = plsc.pack(
                            vs, vs, format=plsc.PackFormat.INTERLEAVED)
                        r = g * 16 + jj
                        for o in range(0, _DH, 32):
                            gb[r, pl.ds(o, 32)] = gb[r, pl.ds(o, 32)] * vb

                for q in range(_KS):
                    for o in range(0, _CH, 16):
                        sidxs[b][q, pl.ds(o, 16)] = cb[_KS + q, pl.ds(o, 16)]
                start_scatter(b)

                @pl.when(ci + 4 < _NCHUNK)
                def _():
                    start_idx(ci + 4, b)

                nb = (b + 2) % 4

                @pl.when(ci + 2 < _NCHUNK)
                def _():
                    @pl.when(ci >= 2)
                    def _():
                        wait_scatter(nb)

                    wait_idx(ci + 2, nb)
                    fix_cols(nb)
                    start_gather(nb)

            return carry

        lax.fori_loop(0, _NCHUNK // 4, quad_body, 0)
        for b in range(4):
            wait_scatter(b)
        plsc.subcore_barrier()

        pltpu.make_async_copy(idtab_src, idtabv, xsems[0]).wait()
        pltpu.make_async_copy(idplain_src, idplainv, xsems[1]).wait()

        obase = c * _BB + s * _BPS
        pairs = (
            (gbufs[0], gbufs[1], gsems[0], gsems[1]),
            (gbufs[2], gbufs[3], gsems[2], gsems[3]),
        )

        def lk_start(j, p):
            tb, ab, tsem, asem = pairs[p]
            pltpu.async_copy(
                table_hbm.at[idtabv.at[j]], tb.at[pl.ds(0, _CH)], tsem)
            pltpu.async_copy(
                acc.at[idplainv.at[j]], ab.at[pl.ds(0, _CH)], asem)

        def lk_wait(j, p):
            tb, ab, tsem, asem = pairs[p]
            pltpu.make_async_copy(
                table_hbm.at[idtabv.at[j]], tb.at[pl.ds(0, _CH)], tsem).wait()
            pltpu.make_async_copy(
                acc.at[idplainv.at[j]], ab.at[pl.ds(0, _CH)], asem).wait()

        half = jnp.bfloat16(0.5)
        lk_start(0, 0)
        for j in range(_BCH):
            if j + 1 < _BCH:
                lk_start(j + 1, (j + 1) % 2)
            lk_wait(j, j % 2)
            tb, ab, _, _ = pairs[j % 2]

            @plsc.parallel_loop(0, _CH, unroll=4)
            def _combine(r):
                for g in range(_DH // 32):
                    sl = pl.ds(g * 32, 32)
                    sm = (tb[r, sl] + ab[r, sl]) * half
                    ev, od = plsc.unpack(
                        sm, format=plsc.PackFormat.INTERLEAVED)
                    fbuf[r, pl.ds(g * 32, 16)] = ev
                    fbuf[r, pl.ds(g * 32 + 16, 16)] = od

            pltpu.sync_copy(fbuf, out_hbm.at[pl.ds(obase + j * _CH, _CH)])

    return k(table, edges_s, idtab_s, idplain_s)


def kernel(u_id, i_id, w_user, w_item, graph_index, graph_values, Wu, bu, Wi, bi):
    u_id = u_id.astype(jnp.int32)
    i_id = i_id.astype(jnp.int32)
    row = graph_index[0].astype(jnp.int32)
    col = graph_index[1].astype(jnp.int32)

    users_emb = _dense_proj(w_user, Wu.T, bu.reshape(1, _D), 1024)
    items_emb = _dense_proj(w_item, Wi.T, bi.reshape(1, _D), 2048)
    all_emb = jnp.concatenate([users_emb, items_emb], axis=0)
    table = all_emb.astype(jnp.bfloat16).reshape(_N * 2, _DH)

    vbits = lax.bitcast_convert_type(
        graph_values.astype(jnp.float32), jnp.int32)
    nch = _NS * _NCHUNK
    packed = jnp.stack([
        col.reshape(nch, _KS, _CH),
        row.reshape(nch, _KS, _CH),
        vbits.reshape(nch, _KS, _CH),
    ], axis=1)
    edges_s = packed.reshape(nch * 3 * _KS, _CH)

    ids = jnp.concatenate([u_id, _NU + i_id])
    idtab_s = (2 * ids[None, :] + jnp.arange(_NC, dtype=jnp.int32)[:, None])
    idtab_s = idtab_s.reshape(_NC * _NS * _BCH, _CH)
    idplain_s = ids.reshape(_NS * _BCH, _CH)

    out = _sc_propagate_and_lookup(table, edges_s, idtab_s, idplain_s)

    halves = out.reshape(_NC, _BB, _DH)
    res = jnp.concatenate([halves[0], halves[1]], axis=1)
    stored = []
    for base in range(0, _D, 32):
        stored.extend(range(base, base + 32, 2))
        stored.extend(range(base + 1, base + 32, 2))
    nat_from_stored = [0] * _D
    for pos, colname in enumerate(stored):
        nat_from_stored[colname] = pos
    res = res[:, jnp.array(nat_from_stored, dtype=jnp.int32)]
    return res[:_B], res[_B:]
